# baseline (speedup 1.0000x reference)
import functools

import jax
import jax.numpy as jnp
from jax import lax
from jax.experimental import pallas as pl
from jax.experimental.pallas import tpu as pltpu
from jax.experimental.pallas import tpu_sc as plsc

D = 64
LN = 50
B = 16384
PAD = 104
NC = 2
NS = 16
NW = NC * NS
PAIRS = B // 2
PPW = PAIRS // NW
BPW = B // NW
P = 4
CHUNKS = PPW // P
NBUF = 2
VPR = D // 16


@functools.partial(
    pl.kernel,
    out_type=jax.ShapeDtypeStruct((B, D), jnp.float32),
    mesh=plsc.VectorSubcoreMesh(core_axis_name="c", subcore_axis_name="s"),
    compiler_params=pltpu.CompilerParams(use_tc_tiling_on_sc=False),
    scratch_types=[
        pltpu.VMEM((PPW, PAD), jnp.int32),
        pltpu.VMEM((NBUF, P * PAD, D), jnp.float32),
        pltpu.VMEM((BPW, D), jnp.float32),
        pltpu.SemaphoreType.DMA,
        pltpu.SemaphoreType.DMA,
    ],
)
def _emb_bag(idx_hbm, w_hbm, out_hbm, idx_v, rows_v, out_v, sem0, sem1):
    wid = lax.axis_index("c") * NS + lax.axis_index("s")
    sems = (sem0, sem1)

    pltpu.sync_copy(idx_hbm.at[pl.ds(wid * PPW, PPW)], idx_v)

    def issue(g, b):
        for j in range(P):
            pltpu.async_copy(
                w_hbm.at[idx_v.at[g * P + j]],
                rows_v.at[b, pl.ds(j * PAD, PAD)],
                sems[b],
            )

    def drain(b):
        for j in range(P):
            pltpu.make_async_copy(
                w_hbm.at[idx_v.at[0]],
                rows_v.at[b, pl.ds(j * PAD, PAD)],
                sems[b],
            ).wait()

    def compute(g, b):
        def bag_body(t, carry):
            base = (t >> 1) * PAD + (t & 1) * LN
            accs = [rows_v[b, base, pl.ds(dd * 16, 16)] for dd in range(VPR)]
            for r in range(1, LN):
                for dd in range(VPR):
                    accs[dd] = accs[dd] + rows_v[b, base + r, pl.ds(dd * 16, 16)]
            bag = g * (2 * P) + t
            for dd in range(VPR):
                out_v[bag, pl.ds(dd * 16, 16)] = accs[dd] * (1.0 / LN)
            return carry
        lax.fori_loop(0, 2 * P, bag_body, 0)

    issue(0, 0)
    issue(1, 1)

    def chunk_pair(g2, carry):
        for b in range(NBUF):
            g = NBUF * g2 + b
            drain(b)
            issue(g + NBUF, b)
        return carry

    lax.fori_loop(0, CHUNKS // NBUF - 1, chunk_pair, 0)
    for b in range(NBUF):
        drain(b)

    pltpu.sync_copy(out_v, out_hbm.at[pl.ds(wid * BPW, BPW)])


def kernel(input_, weight):
    idx = input_.reshape(PAIRS, 2 * LN)
    idx_pad = jnp.pad(idx, ((0, 0), (0, PAD - 2 * LN)))
    return _emb_bag(idx_pad, weight)


# speedup vs baseline: 1.8253x; 1.8253x over previous
import functools

import jax
import jax.numpy as jnp
from jax import lax
from jax.experimental import pallas as pl
from jax.experimental.pallas import tpu as pltpu
from jax.experimental.pallas import tpu_sc as plsc

D = 64
LN = 50
B = 16384
NC = 2
NS = 16
NW = NC * NS
BPW = B // NW
IPW = BPW * LN
CB = 8
CI = CB * LN
IPL = 80
NL = CI // IPL
CHUNKS = BPW // CB
NBUF = 2
VPR = D // 16


@functools.partial(
    pl.kernel,
    out_type=jax.ShapeDtypeStruct((B * D,), jnp.float32),
    mesh=plsc.VectorSubcoreMesh(core_axis_name="c", subcore_axis_name="s"),
    compiler_params=pltpu.CompilerParams(use_tc_tiling_on_sc=False),
    scratch_types=[
        pltpu.VMEM((IPW,), jnp.int32),
        pltpu.VMEM((NBUF, CI, D), jnp.float32),
        pltpu.VMEM((BPW * D,), jnp.float32),
        pltpu.SemaphoreType.DMA,
        pltpu.SemaphoreType.DMA,
    ],
)
def _emb_bag(idx_hbm, w_hbm, out_hbm, idx_v, rows_v, out_v, sem0, sem1):
    wid = lax.axis_index("c") * NS + lax.axis_index("s")
    sems = (sem0, sem1)

    pltpu.sync_copy(idx_hbm.at[pl.ds(wid * IPW, IPW)], idx_v)

    def issue(g, b):
        for j in range(NL):
            pltpu.async_copy(
                w_hbm.at[idx_v.at[pl.ds(g * CI + j * IPL, IPL)]],
                rows_v.at[b, pl.ds(j * IPL, IPL)],
                sems[b],
            )

    def drain(b):
        for j in range(NL):
            pltpu.make_async_copy(
                w_hbm.at[idx_v.at[pl.ds(0, IPL)]],
                rows_v.at[b, pl.ds(j * IPL, IPL)],
                sems[b],
            ).wait()

    def compute(g, b):
        def bag_body(t, carry):
            base = t * LN
            accs = [rows_v[b, base, pl.ds(dd * 16, 16)] for dd in range(VPR)]
            for r in range(1, LN):
                for dd in range(VPR):
                    accs[dd] = accs[dd] + rows_v[b, base + r, pl.ds(dd * 16, 16)]
            obase = (g * CB + t) * D
            for dd in range(VPR):
                out_v[pl.ds(obase + dd * 16, 16)] = accs[dd] * (1.0 / LN)
            return carry
        lax.fori_loop(0, CB, bag_body, 0)

    issue(0, 0)
    issue(1, 1)

    def chunk_pair(g2, carry):
        for b in range(NBUF):
            g = NBUF * g2 + b
            drain(b)
            compute(g, b)
            issue(g + NBUF, b)
        return carry

    lax.fori_loop(0, CHUNKS // NBUF - 1, chunk_pair, 0)
    for b in range(NBUF):
        drain(b)
        compute(CHUNKS - NBUF + b, b)

    pltpu.sync_copy(out_v, out_hbm.at[pl.ds(wid * BPW * D, BPW * D)])


def kernel(input_, weight):
    out = _emb_bag(input_.reshape(-1), weight)
    return out.reshape(B, D)


# speedup vs baseline: 1.8729x; 1.0261x over previous
import functools

import jax
import jax.numpy as jnp
from jax import lax
from jax.experimental import pallas as pl
from jax.experimental.pallas import tpu as pltpu
from jax.experimental.pallas import tpu_sc as plsc

D = 64
LN = 50
B = 16384
NC = 2
NS = 16
NW = NC * NS
BPW = B // NW
IPW = BPW * LN
CB = 8
CI = CB * LN
IPL = 80
NL = CI // IPL
CHUNKS = BPW // CB
NBUF = 2
VPR = D // 16


@functools.partial(
    pl.kernel,
    out_type=jax.ShapeDtypeStruct((B * D,), jnp.float32),
    mesh=plsc.VectorSubcoreMesh(core_axis_name="c", subcore_axis_name="s"),
    compiler_params=pltpu.CompilerParams(use_tc_tiling_on_sc=False),
    scratch_types=[
        pltpu.VMEM((IPW,), jnp.int32),
        pltpu.VMEM((NBUF, CI, D), jnp.float32),
        pltpu.VMEM((BPW * D,), jnp.float32),
        pltpu.SemaphoreType.DMA,
        pltpu.SemaphoreType.DMA,
    ],
)
def _emb_bag(idx_hbm, w_hbm, out_hbm, idx_v, rows_v, out_v, sem0, sem1):
    wid = lax.axis_index("c") * NS + lax.axis_index("s")
    sems = (sem0, sem1)

    pltpu.sync_copy(idx_hbm.at[pl.ds(wid * IPW, IPW)], idx_v)

    def issue(g, b):
        for j in range(NL):
            pltpu.async_copy(
                w_hbm.at[idx_v.at[pl.ds(g * CI + j * IPL, IPL)]],
                rows_v.at[b, pl.ds(j * IPL, IPL)],
                sems[b],
            )

    def drain(b):
        for j in range(NL):
            pltpu.make_async_copy(
                w_hbm.at[idx_v.at[pl.ds(0, IPL)]],
                rows_v.at[b, pl.ds(j * IPL, IPL)],
                sems[b],
            ).wait()

    def compute(g, b):
        def bag_body(t, carry):
            base = t * LN
            accs = [rows_v[b, base, pl.ds(dd * 16, 16)] for dd in range(VPR)]
            for r in range(1, LN):
                for dd in range(VPR):
                    accs[dd] = accs[dd] + rows_v[b, base + r, pl.ds(dd * 16, 16)]
            obase = (g * CB + t) * D
            for dd in range(VPR):
                out_v[pl.ds(obase + dd * 16, 16)] = accs[dd]
            return carry
        lax.fori_loop(0, CB, bag_body, 0)

    issue(0, 0)
    issue(1, 1)

    def chunk_pair(g2, carry):
        for b in range(NBUF):
            g = NBUF * g2 + b
            drain(b)
            compute(g, b)
            issue(g + NBUF, b)
        return carry

    lax.fori_loop(0, CHUNKS // NBUF - 1, chunk_pair, 0)
    for b in range(NBUF):
        drain(b)
        compute(CHUNKS - NBUF + b, b)

    pltpu.sync_copy(out_v, out_hbm.at[pl.ds(wid * BPW * D, BPW * D)])


def kernel(input_, weight):
    flat_idx = jnp.minimum(input_.reshape(-1), weight.shape[0] - 1)
    out = _emb_bag(flat_idx, weight)
    return out.reshape(B, D) * (1.0 / LN)
